# baseline (speedup 1.0000x reference)
import functools

import jax
import jax.numpy as jnp
from jax import lax
from jax.experimental import pallas as pl
from jax.experimental.pallas import tpu as pltpu
from jax.experimental.pallas import tpu_sc as plsc

_B = 16384
_NCAT = 26
_CARD = 100000
_EDIM = 16
_NBIN = 10
_NCONT = 13
_NREST = _NBIN + _NCONT
_GDIM = _NCAT * _EDIM
_H1 = 256
_H2 = 128
_NCLS = 2

_NW = 32
_ROWS_PER_W = _GDIM // _NW
_Q = 4096


def _gather_t(emb_t, idx_t):
    mesh = plsc.VectorSubcoreMesh(core_axis_name="c", subcore_axis_name="s")

    @functools.partial(
        pl.kernel,
        mesh=mesh,
        out_type=jax.ShapeDtypeStruct((_GDIM, _B // 2), jnp.float32),
        scratch_types=[
            pltpu.VMEM((_CARD,), jnp.float32),
            pltpu.VMEM((_B,), jnp.int32),
            pltpu.VMEM((_Q // 2,), jnp.float32),
            pltpu.VMEM((_Q // 2,), jnp.float32),
            pltpu.SemaphoreType.DMA,
            pltpu.SemaphoreType.DMA,
        ],
        compiler_params=pltpu.CompilerParams(needs_layout_passes=False),
    )
    def gather_k(emb_hbm, idx_hbm, out_hbm, rowbuf, idxbuf, outq0, outq1,
                 sem_row, sem_w):
        wid = lax.axis_index("s") * 2 + lax.axis_index("c")

        def do_row(k, cprev):
            r = wid * _ROWS_PER_W + k
            c = r // _EDIM
            e = r % _EDIM
            rcp = pltpu.async_copy(emb_hbm.at[c, e, :], rowbuf, sem_row)

            @pl.when(c != cprev)
            def _():
                pltpu.sync_copy(idx_hbm.at[c, :], idxbuf)

            rcp.wait()
            handles = []
            for h in range(_B // _Q):
                ob = outq0 if h % 2 == 0 else outq1
                if h >= 2:
                    handles[h - 2].wait()

                def gather_quarter(hh, obuf):
                    @plsc.parallel_loop(0, _Q, 32, unroll=4)
                    def _(ii):
                        iv0 = idxbuf[pl.ds(hh * _Q + ii, 16)]
                        iv1 = idxbuf[pl.ds(hh * _Q + ii + 16, 16)]
                        v0 = plsc.load_gather(rowbuf, [iv0])
                        v1 = plsc.load_gather(rowbuf, [iv1])
                        pk = plsc.pack(v0, v1,
                                       format=plsc.PackFormat.INTERLEAVED)
                        obuf[pl.ds(ii // 2, 16)] = plsc.bitcast(
                            pk, jnp.float32)

                gather_quarter(h, ob)
                handles.append(
                    pltpu.async_copy(
                        ob, out_hbm.at[r, pl.ds(h * (_Q // 2), _Q // 2)],
                        sem_w))
            handles[-2].wait()
            handles[-1].wait()
            return c

        lax.fori_loop(0, _ROWS_PER_W, do_row, -1)

    return gather_k(emb_t, idx_t)


def _mlp_body(gath_ref, xre_ref, xro_ref, w1g_ref, w1r_ref, b1_ref, g1_ref,
              be1_ref, w2_ref, b2_ref, g2_ref, be2_ref, w3_ref, b3_ref,
              igp_ref, ibp_ref, oe_ref, oo_ref):
    def ln(h, g, b):
        m = jnp.mean(h, axis=0, keepdims=True)
        v = jnp.mean((h - m) ** 2, axis=0, keepdims=True)
        return g * (h - m) / jnp.sqrt(v + 1e-5) + b

    w = lax.bitcast_convert_type(gath_ref[...], jnp.int32)
    glo = lax.bitcast_convert_type(jnp.left_shift(w, 16), jnp.float32)
    ghi = lax.bitcast_convert_type(
        jnp.bitwise_and(w, jnp.int32(-65536)), jnp.float32)

    def half(g16, xr, o_ref):
        row = lax.broadcasted_iota(jnp.int32, xr.shape, 0)
        binpart = jnp.clip(jnp.round(xr), 0.0, 1.0)
        contpart = xr * igp_ref[...] + ibp_ref[...]
        rest = jnp.where(row < _NBIN, binpart, contpart)
        z1 = (jnp.dot(w1g_ref[...], g16, preferred_element_type=jnp.float32)
              + jnp.dot(w1r_ref[...], rest,
                        preferred_element_type=jnp.float32)
              + b1_ref[...])
        h1 = jnp.maximum(ln(z1, g1_ref[...], be1_ref[...]), 0.0)
        z2 = (jnp.dot(w2_ref[...], h1, preferred_element_type=jnp.float32)
              + b2_ref[...])
        h2 = jnp.maximum(ln(z2, g2_ref[...], be2_ref[...]), 0.0)
        o_ref[...] = (jnp.dot(w3_ref[...], h2,
                              preferred_element_type=jnp.float32)
                      + b3_ref[...])

    half(glo, xre_ref[...], oe_ref)
    half(ghi, xro_ref[...], oo_ref)


_BB = 2048


def _mlp_t(gath_p, xr_e, xr_o, w1g, w1r, b1c, g1c, be1c, w2, b2c, g2c,
           be2c, w3, b3c, igc, ibc):
    const = lambda i: (0, 0)
    hb = _BB // 2
    return pl.pallas_call(
        _mlp_body,
        grid=(_B // _BB,),
        in_specs=[
            pl.BlockSpec((_GDIM, hb), lambda i: (0, i)),
            pl.BlockSpec((_NREST, hb), lambda i: (0, i)),
            pl.BlockSpec((_NREST, hb), lambda i: (0, i)),
            pl.BlockSpec((_H1, _GDIM), const),
            pl.BlockSpec((_H1, _NREST), const),
            pl.BlockSpec((_H1, 1), const),
            pl.BlockSpec((_H1, 1), const),
            pl.BlockSpec((_H1, 1), const),
            pl.BlockSpec((_H2, _H1), const),
            pl.BlockSpec((_H2, 1), const),
            pl.BlockSpec((_H2, 1), const),
            pl.BlockSpec((_H2, 1), const),
            pl.BlockSpec((_NCLS, _H2), const),
            pl.BlockSpec((_NCLS, 1), const),
            pl.BlockSpec((_NREST, 1), const),
            pl.BlockSpec((_NREST, 1), const),
        ],
        out_specs=[
            pl.BlockSpec((_NCLS, hb), lambda i: (0, i)),
            pl.BlockSpec((_NCLS, hb), lambda i: (0, i)),
        ],
        out_shape=[
            jax.ShapeDtypeStruct((_NCLS, _B // 2), jnp.float32),
            jax.ShapeDtypeStruct((_NCLS, _B // 2), jnp.float32),
        ],
        compiler_params=pltpu.CompilerParams(
            dimension_semantics=("arbitrary",)),
    )(gath_p, xr_e, xr_o, w1g, w1r, b1c, g1c, be1c, w2, b2c, g2c, be2c, w3,
      b3c, igc, ibc)


def kernel(x, emb, W1, b1, g1, be1, W2, b2, g2, be2, W3, b3, in_gamma,
           in_beta):
    idx_t = jnp.clip(jnp.round(x[:, :_NCAT]), 0, _CARD - 1).astype(
        jnp.int32).T
    idx_t = idx_t.reshape(_NCAT, _B // 32, 16, 2).transpose(
        0, 1, 3, 2).reshape(_NCAT, _B)
    emb_t = jnp.transpose(emb, (0, 2, 1))
    gath_t = _gather_t(emb_t, idx_t)

    xr_t = x[:, _NCAT:].T
    xr_e = xr_t[:, 0::2]
    xr_o = xr_t[:, 1::2]
    igc = jnp.concatenate(
        [jnp.ones((_NBIN,), jnp.float32), in_gamma / (1.0 + 1e-6)]
    ).reshape(_NREST, 1)
    ibc = jnp.concatenate(
        [jnp.zeros((_NBIN,), jnp.float32), in_beta]
    ).reshape(_NREST, 1)

    out_e, out_o = _mlp_t(
        gath_t, xr_e, xr_o,
        W1[:, :_GDIM], W1[:, _GDIM:],
        b1.reshape(_H1, 1), g1.reshape(_H1, 1), be1.reshape(_H1, 1),
        W2, b2.reshape(_H2, 1), g2.reshape(_H2, 1), be2.reshape(_H2, 1),
        W3, b3.reshape(_NCLS, 1),
        igc, ibc,
    )
    out_t = jnp.stack([out_e, out_o], axis=-1).reshape(_NCLS, _B)
    return out_t.T


# speedup vs baseline: 1.2670x; 1.2670x over previous
import functools

import jax
import jax.numpy as jnp
from jax import lax
from jax.experimental import pallas as pl
from jax.experimental.pallas import tpu as pltpu
from jax.experimental.pallas import tpu_sc as plsc

_B = 16384
_NCAT = 26
_CARD = 100000
_EDIM = 16
_NBIN = 10
_NCONT = 13
_NREST = _NBIN + _NCONT
_GDIM = _NCAT * _EDIM
_H1 = 256
_H2 = 128
_NCLS = 2

_NW = 32
_ROWS_PER_W = _GDIM // _NW
_Q = 4096


def _gather_t(emb_t, idx_t):
    mesh = plsc.VectorSubcoreMesh(core_axis_name="c", subcore_axis_name="s")

    @functools.partial(
        pl.kernel,
        mesh=mesh,
        out_type=jax.ShapeDtypeStruct((_GDIM, _B), jnp.float32),
        scratch_types=[
            pltpu.VMEM((_CARD,), jnp.float32),
            pltpu.VMEM((_B,), jnp.int32),
            pltpu.VMEM((_Q,), jnp.float32),
            pltpu.VMEM((_Q,), jnp.float32),
            pltpu.SemaphoreType.DMA,
            pltpu.SemaphoreType.DMA,
        ],
        compiler_params=pltpu.CompilerParams(needs_layout_passes=False),
    )
    def gather_k(emb_hbm, idx_hbm, out_hbm, rowbuf, idxbuf, outq0, outq1,
                 sem_row, sem_w):
        wid = lax.axis_index("s") * 2 + lax.axis_index("c")

        def do_row(k, cprev):
            r = wid * _ROWS_PER_W + k
            c = r // _EDIM
            e = r % _EDIM
            rcp = pltpu.async_copy(emb_hbm.at[c, e, :], rowbuf, sem_row)

            @pl.when(c != cprev)
            def _():
                pltpu.sync_copy(idx_hbm.at[c, :], idxbuf)

            rcp.wait()
            handles = []
            for h in range(_B // _Q):
                ob = outq0 if h % 2 == 0 else outq1
                if h >= 2:
                    handles[h - 2].wait()

                def gather_quarter(hh, obuf):
                    @plsc.parallel_loop(0, _Q, 16, unroll=8)
                    def _(ii):
                        iv = idxbuf[pl.ds(hh * _Q + ii, 16)]
                        obuf[pl.ds(ii, 16)] = plsc.load_gather(rowbuf, [iv])

                gather_quarter(h, ob)
                handles.append(
                    pltpu.async_copy(ob, out_hbm.at[r, pl.ds(h * _Q, _Q)],
                                     sem_w))
            handles[-2].wait()
            handles[-1].wait()
            return c

        lax.fori_loop(0, _ROWS_PER_W, do_row, -1)

    return gather_k(emb_t, idx_t)


def _mlp_body(gath_ref, xr_ref, w1g_ref, w1r_ref, b1_ref, g1_ref, be1_ref,
              w2_ref, b2_ref, g2_ref, be2_ref, w3_ref, b3_ref, igp_ref,
              ibp_ref, o_ref):
    def ln(h, g, b):
        m = jnp.mean(h, axis=0, keepdims=True)
        v = jnp.mean((h - m) ** 2, axis=0, keepdims=True)
        return g * (h - m) / jnp.sqrt(v + 1e-5) + b

    xr = xr_ref[...]
    row = lax.broadcasted_iota(jnp.int32, xr.shape, 0)
    binpart = jnp.clip(jnp.round(xr), 0.0, 1.0)
    contpart = xr * igp_ref[...] + ibp_ref[...]
    rest = jnp.where(row < _NBIN, binpart, contpart)
    z1 = (jnp.dot(w1g_ref[...], gath_ref[...], preferred_element_type=jnp.float32)
          + jnp.dot(w1r_ref[...], rest, preferred_element_type=jnp.float32)
          + b1_ref[...])
    h1 = jnp.maximum(ln(z1, g1_ref[...], be1_ref[...]), 0.0)
    z2 = jnp.dot(w2_ref[...], h1, preferred_element_type=jnp.float32) + b2_ref[...]
    h2 = jnp.maximum(ln(z2, g2_ref[...], be2_ref[...]), 0.0)
    o_ref[...] = (jnp.dot(w3_ref[...], h2, preferred_element_type=jnp.float32)
                  + b3_ref[...])


_BB = 2048


def _mlp_t(gath_t, xr_t, w1g, w1r, b1c, g1c, be1c, w2, b2c, g2c, be2c, w3,
           b3c, igc, ibc):
    const = lambda i: (0, 0)
    return pl.pallas_call(
        _mlp_body,
        grid=(_B // _BB,),
        in_specs=[
            pl.BlockSpec((_GDIM, _BB), lambda i: (0, i)),
            pl.BlockSpec((_NREST, _BB), lambda i: (0, i)),
            pl.BlockSpec((_H1, _GDIM), const),
            pl.BlockSpec((_H1, _NREST), const),
            pl.BlockSpec((_H1, 1), const),
            pl.BlockSpec((_H1, 1), const),
            pl.BlockSpec((_H1, 1), const),
            pl.BlockSpec((_H2, _H1), const),
            pl.BlockSpec((_H2, 1), const),
            pl.BlockSpec((_H2, 1), const),
            pl.BlockSpec((_H2, 1), const),
            pl.BlockSpec((_NCLS, _H2), const),
            pl.BlockSpec((_NCLS, 1), const),
            pl.BlockSpec((_NREST, 1), const),
            pl.BlockSpec((_NREST, 1), const),
        ],
        out_specs=pl.BlockSpec((_NCLS, _BB), lambda i: (0, i)),
        out_shape=jax.ShapeDtypeStruct((_NCLS, _B), jnp.float32),
        compiler_params=pltpu.CompilerParams(
            dimension_semantics=("arbitrary",)),
    )(gath_t, xr_t, w1g, w1r, b1c, g1c, be1c, w2, b2c, g2c, be2c, w3, b3c,
      igc, ibc)


def kernel(x, emb, W1, b1, g1, be1, W2, b2, g2, be2, W3, b3, in_gamma,
           in_beta):
    idx_t = jnp.clip(jnp.round(x[:, :_NCAT]), 0, _CARD - 1).astype(
        jnp.int32).T
    emb_t = jnp.transpose(emb, (0, 2, 1))
    gath_t = _gather_t(emb_t, idx_t)

    xr_t = x[:, _NCAT:].T
    igc = jnp.concatenate(
        [jnp.ones((_NBIN,), jnp.float32), in_gamma / (1.0 + 1e-6)]
    ).reshape(_NREST, 1)
    ibc = jnp.concatenate(
        [jnp.zeros((_NBIN,), jnp.float32), in_beta]
    ).reshape(_NREST, 1)

    out_t = _mlp_t(
        gath_t, xr_t,
        W1[:, :_GDIM], W1[:, _GDIM:],
        b1.reshape(_H1, 1), g1.reshape(_H1, 1), be1.reshape(_H1, 1),
        W2, b2.reshape(_H2, 1), g2.reshape(_H2, 1), be2.reshape(_H2, 1),
        W3, b3.reshape(_NCLS, 1),
        igc, ibc,
    )
    return out_t.T
